# Initial kernel scaffold; baseline (speedup 1.0000x reference)
import jax
import jax.numpy as jnp
from jax.experimental import pallas as pl


def kernel(x, edge_index, batch, W1, as1, ad1, b1, W2, as2, ad2, b2, W3, as3, ad3, b3, lin1_W, lin1_b, lin2_W, lin2_b):
    raise NotImplementedError("write your pallas kernel here")


# speedup vs baseline: 18.7270x; 18.7270x over previous
import jax
import jax.numpy as jnp
from jax import lax
from jax.experimental import pallas as pl
from jax.experimental.pallas import tpu as pltpu
from jax.experimental.pallas import tpu_sc as plsc

N = 10000
NP = 10240
E = 320000
H = 128
B = 64
C = 10

NC = 2
NS = 16
NW = NC * NS
L = 16

K = 80
TPC = E // K // NW
HALF = NP // 2
NQ = 4
QTR = NP // NQ
RPQ = QTR // NS
RPW = NP // NW
CAP = 5120
CR = CAP // K
GP = 72
SENT = NP - L

_mesh = plsc.VectorSubcoreMesh(
    core_axis_name="c", subcore_axis_name="s", num_cores=NC, num_subcores=NS)
_scparams = pltpu.CompilerParams(needs_layout_passes=False)


def _zvec():
  return jnp.zeros((L,), jnp.float32)


def _bucket_body(src2, dst2, bs_out, bd_out, cnt_out,
                 src_v, dst_v, b_s, b_d, cnt_v):
  cid = lax.axis_index("c")
  sid = lax.axis_index("s")
  wid = sid * NC + cid

  pltpu.sync_copy(src2.at[wid], src_v)
  pltpu.sync_copy(dst2.at[wid], dst_v)

  def chunk(j, carry):
    os = list(carry)
    for c5 in range(K // L):
      sv = src_v[j, pl.ds(c5 * L, L)]
      dv = dst_v[j, pl.ds(c5 * L, L)]
      qv = dv // QTR
      for g in range(NQ):
        m = qv == g
        plsc.store_compressed(b_s.at[g, pl.ds(os[g], L)], sv, mask=m)
        plsc.store_compressed(b_d.at[g, pl.ds(os[g], L)], dv - g * QTR,
                              mask=m)
        os[g] = jnp.minimum(
            os[g] + jnp.max(plsc.all_reduce_population_count(m)), CAP - K)
    return tuple(os)
  os = lax.fori_loop(0, TPC, chunk, (jnp.int32(0),) * NQ)

  sent = jnp.full((L,), SENT, jnp.int32)
  zero = jnp.zeros((L,), jnp.int32)
  for g in range(NQ):
    for t5 in range(K // L):
      b_s[g, pl.ds(os[g] + t5 * L, L)] = sent
      b_d[g, pl.ds(os[g] + t5 * L, L)] = zero

  lanes = lax.iota(jnp.int32, L)
  cv = jnp.zeros((L,), jnp.int32)
  for g in range(NQ):
    cv = cv + jnp.where(lanes == g, os[g], 0)
  cnt_v[pl.ds(0, L)] = cv

  for g in range(NQ):
    pltpu.sync_copy(b_s.at[g], bs_out.at[wid, g])
    pltpu.sync_copy(b_d.at[g], bd_out.at[wid, g])
  pltpu.sync_copy(cnt_v, cnt_out.at[wid, 0])


_bucket_call = pl.kernel(
    _bucket_body,
    out_type=[jax.ShapeDtypeStruct((NW, NQ, CAP), jnp.int32),
              jax.ShapeDtypeStruct((NW, NQ, CAP), jnp.int32),
              jax.ShapeDtypeStruct((NW, 1, L), jnp.int32)],
    mesh=_mesh,
    compiler_params=_scparams,
    scratch_types=[
        pltpu.VMEM((TPC, K), jnp.int32),
        pltpu.VMEM((TPC, K), jnp.int32),
        pltpu.VMEM((NQ, CAP), jnp.int32),
        pltpu.VMEM((NQ, CAP), jnp.int32),
        pltpu.VMEM((L,), jnp.int32),
    ],
)


def _edge_body(bsrc, bdst, counts, asrc, adst, h, sp_out, tl_out,
               bs0_v, bs1_v, bd0_v, bd1_v, cnt_v, asrc_v, adst_v, s_v,
               exb_v, rows_v, zrow_v, out_acc):
  cid = lax.axis_index("c")
  sid = lax.axis_index("s")
  wid = sid * NC + cid

  for u in range(2):
    pltpu.sync_copy(counts.at[2 * sid + u, 0], cnt_v.at[u])
  pltpu.sync_copy(asrc, asrc_v)
  pltpu.sync_copy(adst, adst_v)
  asrc_v[pl.ds(SENT, L)] = jnp.full((L,), -1e30, jnp.float32)

  def zs(i, carry):
    s_v[pl.ds(i * L, L)] = _zvec()
    return carry
  lax.fori_loop(0, HALF // L, zs, 0)

  def zr(i, carry):
    for c8 in range(H // L):
      zrow_v[i, pl.ds(c8 * L, L)] = _zvec()
    return carry
  lax.fori_loop(0, K, zr, 0)

  lanes = lax.iota(jnp.int32, L)
  base = cid * HALF

  def qpass(q, qcarry):
    g = 2 * cid + q
    pltpu.sync_copy(bsrc.at[2 * sid, g], bs0_v)
    pltpu.sync_copy(bdst.at[2 * sid, g], bd0_v)
    pltpu.sync_copy(bsrc.at[2 * sid + 1, g], bs1_v)
    pltpu.sync_copy(bdst.at[2 * sid + 1, g], bd1_v)
    for i in range(RPQ // K):
      pltpu.sync_copy(zrow_v, out_acc.at[pl.ds(sid * RPQ + i * K, K)])
    plsc.subcore_barrier()

    for u, bsu, bdu in ((0, bs0_v, bd0_v), (1, bs1_v, bd1_v)):
      cnt = jnp.max(plsc.load_gather(
          cnt_v, [jnp.full((L,), u, jnp.int32), jnp.full((L,), g, jnp.int32)]))
      nch = jnp.minimum((cnt + K - 1) // K, CR)

      def chunk(j, carry, bsu=bsu, bdu=bdu):
        for c5 in range(K // L):
          sv = jnp.clip(bsu[j, pl.ds(c5 * L, L)], 0, NP - 1)
          dv = jnp.clip(bdu[j, pl.ds(c5 * L, L)], 0, QTR - 1)
          bsu[j, pl.ds(c5 * L, L)] = sv
          bdu[j, pl.ds(c5 * L, L)] = dv
          e = (plsc.load_gather(asrc_v, [sv])
               + plsc.load_gather(adst_v, [dv + (base + q * QTR)]))
          e = jnp.maximum(e, e * 0.2)
          ex = jnp.exp(e)
          exb_v[pl.ds(c5 * L, L)] = ex
          plsc.addupdate_scatter(s_v, [dv + q * QTR], ex)
        pltpu.sync_copy(h.at[bsu.at[j]], rows_v)

        def rowf(r, rcarry):
          exr = plsc.load_gather(exb_v, [jnp.full((L,), r, jnp.int32)])
          for c8 in range(H // L):
            rows_v[r, pl.ds(c8 * L, L)] = rows_v[r, pl.ds(c8 * L, L)] * exr
          return rcarry
        lax.fori_loop(0, K, rowf, 0)
        pltpu.sync_copy(rows_v, out_acc.at[bdu.at[j]], add=True)
        return carry
      lax.fori_loop(0, nch, chunk, 0)

    plsc.subcore_barrier()
    rbase = base + q * QTR + sid * RPQ
    for i in range(RPQ // K):
      pltpu.sync_copy(out_acc.at[pl.ds(sid * RPQ + i * K, K)], rows_v)
      pltpu.sync_copy(rows_v, tl_out.at[pl.ds(rbase + i * K, K)])
    return qcarry
  lax.fori_loop(0, 2, qpass, 0)

  pltpu.sync_copy(s_v, sp_out.at[wid, 0])


_edge_call = pl.kernel(
    _edge_body,
    out_type=[jax.ShapeDtypeStruct((NW, 1, HALF), jnp.float32),
              jax.ShapeDtypeStruct((NP, H), jnp.float32)],
    mesh=_mesh,
    compiler_params=_scparams,
    scratch_types=[
        pltpu.VMEM((CR, K), jnp.int32),
        pltpu.VMEM((CR, K), jnp.int32),
        pltpu.VMEM((CR, K), jnp.int32),
        pltpu.VMEM((CR, K), jnp.int32),
        pltpu.VMEM((2, L), jnp.int32),
        pltpu.VMEM((NP,), jnp.float32),
        pltpu.VMEM((NP,), jnp.float32),
        pltpu.VMEM((HALF,), jnp.float32),
        pltpu.VMEM((K,), jnp.float32),
        pltpu.VMEM((K, H), jnp.float32),
        pltpu.VMEM((K, H), jnp.float32),
        pltpu.VMEM_SHARED((QTR, H), jnp.float32),
    ],
)


def _pool_body(tl, sparts, b3, batchp, pp_out,
               bt_v, s_tmp, s_v, t_v, b_v, acc):
  cid = lax.axis_index("c")
  sid = lax.axis_index("s")
  wid = sid * NC + cid
  rb = wid * RPW
  hc = wid // NS
  kk = wid % NS

  pltpu.sync_copy(batchp.at[wid, 0], bt_v)
  pltpu.sync_copy(b3, b_v)

  def cps(t, carry):
    pltpu.sync_copy(sparts.at[2 * t + hc, kk, 0], s_tmp.at[t])
    return carry
  lax.fori_loop(0, NS, cps, 0)

  def sums(k, carry):
    acc16 = s_tmp[0, pl.ds(k * L, L)]
    for t in range(1, NS):
      acc16 = acc16 + s_tmp[t, pl.ds(k * L, L)]
    s_v[pl.ds(k * L, L)] = acc16
    return carry
  lax.fori_loop(0, RPW // L, sums, 0)

  def za(i, carry):
    for c8 in range(H // L):
      acc[i, pl.ds(c8 * L, L)] = jnp.full((L,), -jnp.inf, jnp.float32)
    return carry
  lax.fori_loop(0, GP, za, 0)

  lanes = lax.iota(jnp.int32, L)
  for i in range(RPW // K):
    pltpu.sync_copy(tl.at[wid * (RPW // K) + i], t_v)

    def rowf(r, rcarry):
      ridx = jnp.full((L,), i * K, jnp.int32) + jnp.full((L,), r, jnp.int32)
      gv = plsc.load_gather(bt_v, [ridx])
      sv = plsc.load_gather(s_v, [ridx])
      inv = 1.0 / (sv + 1e-16)
      for c8 in range(H // L):
        v = t_v[r, pl.ds(c8 * L, L)] * inv
        v = jnp.maximum(v + b_v[pl.ds(c8 * L, L)], 0.0)
        colidx = lanes + (c8 * L)
        a = plsc.load_gather(acc, [gv, colidx])
        plsc.store_scatter(acc, [gv, colidx], jnp.maximum(a, v))
      return rcarry
    lax.fori_loop(0, K, rowf, 0)

  pltpu.sync_copy(acc, pp_out.at[wid])


_pool_call = pl.kernel(
    _pool_body,
    out_type=[jax.ShapeDtypeStruct((NW, GP, H), jnp.float32)],
    mesh=_mesh,
    compiler_params=_scparams,
    scratch_types=[
        pltpu.VMEM((RPW,), jnp.int32),
        pltpu.VMEM((NS, RPW), jnp.float32),
        pltpu.VMEM((RPW,), jnp.float32),
        pltpu.VMEM((K, H), jnp.float32),
        pltpu.VMEM((H,), jnp.float32),
        pltpu.VMEM((GP, H), jnp.float32),
    ],
)


RB = 1024
NHB = HALF // RB


def _mm1_body(x_ref, w_ref, a_ref, h_ref, ao_ref):
  h = jnp.dot(x_ref[...], w_ref[...], preferred_element_type=jnp.float32, precision=lax.Precision.HIGHEST)
  h_ref[...] = h
  ao_ref[...] = jnp.dot(h, a_ref[...], preferred_element_type=jnp.float32, precision=lax.Precision.HIGHEST)


def _mm1(xp, w, a):
  return pl.pallas_call(
      _mm1_body,
      grid=(NP // RB,),
      in_specs=[
          pl.BlockSpec((RB, H), lambda i: (i, 0)),
          pl.BlockSpec((H, H), lambda i: (0, 0)),
          pl.BlockSpec((H, H), lambda i: (0, 0)),
      ],
      out_specs=[
          pl.BlockSpec((RB, H), lambda i: (i, 0)),
          pl.BlockSpec((RB, H), lambda i: (i, 0)),
      ],
      out_shape=[
          jax.ShapeDtypeStruct((NP, H), jnp.float32),
          jax.ShapeDtypeStruct((NP, H), jnp.float32),
      ],
  )(xp, w, a)


def _mm2_body(t_ref, sp_ref, b_ref, w_ref, a_ref, h_ref, ao_ref):
  i = pl.program_id(0)
  half = i // NHB
  rowmask = (lax.broadcasted_iota(jnp.int32, (NW, 1, RB), 0) % 2) == half
  ssum = jnp.sum(jnp.where(rowmask, sp_ref[...], 0.0), axis=(0, 1))
  inv = 1.0 / (ssum + 1e-16)
  x = t_ref[...] * inv[:, None] + b_ref[...]
  x = jnp.maximum(x, 0.0)
  h = jnp.dot(x, w_ref[...], preferred_element_type=jnp.float32, precision=lax.Precision.HIGHEST)
  h_ref[...] = h
  ao_ref[...] = jnp.dot(h, a_ref[...], preferred_element_type=jnp.float32, precision=lax.Precision.HIGHEST)


def _mm2(tl, sp, b, w, a):
  return pl.pallas_call(
      _mm2_body,
      grid=(NP // RB,),
      in_specs=[
          pl.BlockSpec((RB, H), lambda i: (i, 0)),
          pl.BlockSpec((NW, 1, RB), lambda i: (0, 0, i % NHB)),
          pl.BlockSpec((1, H), lambda i: (0, 0)),
          pl.BlockSpec((H, H), lambda i: (0, 0)),
          pl.BlockSpec((H, H), lambda i: (0, 0)),
      ],
      out_specs=[
          pl.BlockSpec((RB, H), lambda i: (i, 0)),
          pl.BlockSpec((RB, H), lambda i: (i, 0)),
      ],
      out_shape=[
          jax.ShapeDtypeStruct((NP, H), jnp.float32),
          jax.ShapeDtypeStruct((NP, H), jnp.float32),
      ],
  )(tl, sp, b.reshape(1, H), w, a)


def _head_body(pp_ref, w1_ref, b1_ref, w2_ref, b2_ref, o_ref):
  p = jnp.max(pp_ref[...], axis=0)[:B]
  p = jnp.where(jnp.isfinite(p), p, 0.0)
  z = jnp.dot(p, w1_ref[...], preferred_element_type=jnp.float32, precision=lax.Precision.HIGHEST) + b1_ref[...]
  z = jnp.maximum(z, 0.0)
  z = jnp.dot(z, w2_ref[...], preferred_element_type=jnp.float32, precision=lax.Precision.HIGHEST) + b2_ref[...]
  colmask = lax.broadcasted_iota(jnp.int32, (B, H), 1) < C
  zm = jnp.where(colmask, z, -1e30)
  m = jnp.max(zm, axis=1, keepdims=True)
  lse = jnp.log(jnp.sum(jnp.exp(zm - m), axis=1, keepdims=True)) + m
  o_ref[...] = zm - lse


def _head(pp, w1, b1, w2p, b2p):
  return pl.pallas_call(
      _head_body,
      in_specs=[
          pl.BlockSpec((NW, GP, H), lambda: (0, 0, 0)),
          pl.BlockSpec((H, H), lambda: (0, 0)),
          pl.BlockSpec((1, H), lambda: (0, 0)),
          pl.BlockSpec((H, H), lambda: (0, 0)),
          pl.BlockSpec((1, H), lambda: (0, 0)),
      ],
      out_specs=pl.BlockSpec((B, H), lambda: (0, 0)),
      out_shape=jax.ShapeDtypeStruct((B, H), jnp.float32),
  )(pp, w1, b1.reshape(1, H), w2p, b2p.reshape(1, H))


def kernel(x, edge_index, batch, W1, as1, ad1, b1, W2, as2, ad2, b2,
           W3, as3, ad3, b3, lin1_W, lin1_b, lin2_W, lin2_b):
  src2 = edge_index[0].reshape(NW, TPC, K)
  dst2 = edge_index[1].reshape(NW, TPC, K)
  xp = jnp.pad(x, ((0, NP - N), (0, 0)))
  batchp = jnp.concatenate(
      [batch, jnp.full((NP - N,), B, jnp.int32)]).astype(jnp.int32)

  bs, bd, cnts = _bucket_call(src2, dst2)
  bs4 = bs.reshape(NW, NQ, CR, K)
  bd4 = bd.reshape(NW, NQ, CR, K)

  def amat(a_s, a_d):
    m = jnp.zeros((H, H), jnp.float32)
    return m.at[:, 0].set(a_s).at[:, 1].set(a_d)

  def edge(asrc, adst, h):
    return _edge_call(bs4, bd4, cnts, asrc, adst, h)

  h1, a1 = _mm1(xp, W1, amat(as1, ad1))

  wst = jnp.stack([W2, W3, W3])
  ast = jnp.stack([amat(as2, ad2), amat(as3, ad3), amat(as3, ad3)])
  bst = jnp.stack([b1, b2, b3])

  def step(carry, i):
    h, am, _, _ = carry
    s, t = edge(am[:, 0], am[:, 1], h)
    hn, amn = _mm2(t, s, bst[i], wst[i], ast[i])
    return (hn, amn, s, t), None

  init = (h1, a1, jnp.zeros((NW, 1, HALF), jnp.float32),
          jnp.zeros((NP, H), jnp.float32))
  (_, _, s3, t3), _ = lax.scan(step, init, jnp.arange(3))
  pp = _pool_call(t3.reshape(NP // K, K, H), s3.reshape(NW, NS, 1, RPW), b3,
                  batchp.reshape(NW, 1, RPW))[0]

  w2p = jnp.zeros((H, H), jnp.float32).at[:, :C].set(lin2_W)
  b2p = jnp.zeros((H,), jnp.float32).at[:C].set(lin2_b)
  o = _head(pp, lin1_W, lin1_b, w2p, b2p)
  return o[:, :C]
